# baseline (speedup 1.0000x reference)
import functools

import jax
import jax.numpy as jnp
from jax import lax
from jax.experimental import pallas as pl
from jax.experimental.pallas import tpu as pltpu
from jax.experimental.pallas import tpu_sc as plsc

N = 10000
E = 320000
D_IN = 128
D_H = 128
D_OUT = 64

NC = 2
NS = 16

CHUNK = 128
CPT = 160
CPT2 = CPT // 2
EPAD = NS * CPT * CHUNK
NROW = 10112
RPT = NROW // NS
RCH = (128, 128, 128, 128, 120)

BS = 400
NB = N // BS
GRID = 2 * NB


def _zero_acc(z_hbm, stage_v, acc_sh, s):
    pltpu.sync_copy(z_hbm, stage_v)
    off = 0
    for sz in RCH:
        pltpu.sync_copy(stage_v.at[pl.ds(0, sz)],
                        acc_sh.at[pl.ds(s * RPT + off, sz)])
        off += sz


def _copy_out(acc_sh, stage_v, o_hbm, c, s):
    off = 0
    for sz in RCH:
        pltpu.sync_copy(acc_sh.at[pl.ds(s * RPT + off, sz)],
                        stage_v.at[pl.ds(0, sz)])
        pltpu.sync_copy(stage_v.at[pl.ds(0, sz)],
                        o_hbm.at[c, pl.ds(s * RPT + off, sz)])
        off += sz


IB = 8


def _make_agg(chunks, d):
    mesh = plsc.VectorSubcoreMesh(core_axis_name="c", subcore_axis_name="s")
    scratch = [
        pltpu.VMEM((IB, CHUNK), jnp.int32),
        pltpu.VMEM((IB, CHUNK), jnp.int32),
        pltpu.VMEM((CHUNK, d), jnp.float32),
        pltpu.VMEM((CHUNK, d), jnp.float32),
        pltpu.VMEM_SHARED((NROW, d), jnp.float32),
        pltpu.SemaphoreType.DMA,
        pltpu.SemaphoreType.DMA,
    ]

    @functools.partial(pl.kernel,
                       out_type=jax.ShapeDtypeStruct((NC, NROW, d),
                                                     jnp.float32),
                       mesh=mesh, scratch_types=scratch)
    def agg(p_hbm, src_hbm, dst_hbm, zrow_hbm, s_hbm,
            src_blk, dst_blk, rows0, rows1, acc_sh, sem0, sem1):
        c = lax.axis_index("c")
        s = lax.axis_index("s")
        rows = (rows0, rows1)
        sems = (sem0, sem1)

        _zero_acc(zrow_hbm, rows0, acc_sh, s)
        plsc.subcore_barrier()

        def outer(ob, carry):
            pltpu.sync_copy(src_hbm.at[c, s, ob], src_blk)
            pltpu.sync_copy(dst_hbm.at[c, s, ob], dst_blk)
            cps = [pltpu.async_copy(p_hbm.at[src_blk.at[0]], rows[0],
                                    sems[0])]
            for j in range(1, IB):
                b = j & 1
                cps.append(pltpu.async_copy(p_hbm.at[src_blk.at[j]],
                                            rows[b], sems[b]))
                cps[j - 1].wait()
                pltpu.sync_copy(rows[(j - 1) & 1],
                                acc_sh.at[dst_blk.at[j - 1]], add=True)
            cps[IB - 1].wait()
            pltpu.sync_copy(rows[(IB - 1) & 1],
                            acc_sh.at[dst_blk.at[IB - 1]], add=True)
            return carry

        lax.fori_loop(0, chunks // IB, outer, 0)
        plsc.subcore_barrier()

        _copy_out(acc_sh, rows0, s_hbm, c, s)

    return agg


def _make_cnt():
    mesh = plsc.VectorSubcoreMesh(core_axis_name="c", subcore_axis_name="s")
    scratch = [
        pltpu.VMEM((IB, CHUNK), jnp.int32),
        pltpu.VMEM((CHUNK, D_H), jnp.float32),
        pltpu.VMEM_SHARED((NROW, D_H), jnp.float32),
        pltpu.SemaphoreType.DMA,
    ]

    @functools.partial(pl.kernel,
                       out_type=jax.ShapeDtypeStruct((NC, NROW, D_H),
                                                     jnp.float32),
                       mesh=mesh, scratch_types=scratch)
    def cntk(dst_hbm, zrow_hbm, one_hbm, cnt_hbm, dst_blk, ones_v, acc_sh,
             sem):
        c = lax.axis_index("c")
        s = lax.axis_index("s")

        _zero_acc(zrow_hbm, ones_v, acc_sh, s)
        pltpu.sync_copy(one_hbm, ones_v)
        plsc.subcore_barrier()

        def outer(ob, carry):
            pltpu.sync_copy(dst_hbm.at[c, s, ob], dst_blk)
            cps = [pltpu.async_copy(ones_v, acc_sh.at[dst_blk.at[j]], sem,
                                    add=True)
                   for j in range(IB)]
            for cp in cps:
                cp.wait()
            return carry

        lax.fori_loop(0, CPT2 // IB, outer, 0)
        plsc.subcore_barrier()

        _copy_out(acc_sh, ones_v, cnt_hbm, c, s)

    return cntk


_agg01 = _make_agg(CPT, D_H)
_agg2 = _make_agg(CPT2, D_H)
_cnt = _make_cnt()


def _pre_body(x_ref, n_ref, wl_ref, wr_ref, bl_ref, p_ref, q_ref):
    b = pl.program_id(0)
    xb = x_ref[...]
    nb = n_ref[...]
    nn = jnp.sqrt(jnp.sum(nb * nb, axis=1, keepdims=True))
    noisy = xb + jnp.sign(xb) * (nb / jnp.maximum(nn, 1e-12)) * 0.3
    xx = jnp.where(b >= NB, noisy, xb)
    p_ref[...] = jnp.dot(xx, wl_ref[...], preferred_element_type=jnp.float32)
    q_ref[...] = (jnp.dot(xx, wr_ref[...], preferred_element_type=jnp.float32)
                  + bl_ref[...])


def _tc_pre(x, noise, wlt, wrt, bl):
    return pl.pallas_call(
        _pre_body,
        grid=(GRID,),
        in_specs=[
            pl.BlockSpec((BS, D_IN), lambda b: (b % NB, 0)),
            pl.BlockSpec((BS, D_IN), lambda b: (b % NB, 0)),
            pl.BlockSpec((D_IN, D_H), lambda b: (0, 0)),
            pl.BlockSpec((D_IN, D_H), lambda b: (0, 0)),
            pl.BlockSpec((1, D_H), lambda b: (0, 0)),
        ],
        out_specs=[
            pl.BlockSpec((BS, D_H), lambda b: (b, 0)),
            pl.BlockSpec((BS, D_H), lambda b: (b, 0)),
        ],
        out_shape=[
            jax.ShapeDtypeStruct((2 * N, D_H), jnp.float32),
            jax.ShapeDtypeStruct((2 * N, D_H), jnp.float32),
        ],
    )(x, noise, wlt, wrt, bl)


def _mid_body(s_ref, c0_ref, c1_ref, q_ref, wl_ref, wr_ref, bl_ref,
              h_ref, p_ref, q2_ref):
    cnt = c0_ref[0][:, :1] + c1_ref[0][:, :1]
    recip = 1.0 / jnp.maximum(cnt, 1.0)
    h = jnp.maximum(s_ref[0] * recip + q_ref[...], 0.0)
    h_ref[...] = h
    p_ref[...] = jnp.dot(h, wl_ref[...], preferred_element_type=jnp.float32)
    q2_ref[...] = (jnp.dot(h, wr_ref[...], preferred_element_type=jnp.float32)
                   + bl_ref[...])


def _tc_mid(seg, cnt, q, wlt, wrt, bl):
    return pl.pallas_call(
        _mid_body,
        grid=(GRID,),
        in_specs=[
            pl.BlockSpec((1, BS, D_H), lambda b: (b // NB, b % NB, 0)),
            pl.BlockSpec((1, BS, D_H), lambda b: (0, b % NB, 0)),
            pl.BlockSpec((1, BS, D_H), lambda b: (1, b % NB, 0)),
            pl.BlockSpec((BS, D_H), lambda b: (b, 0)),
            pl.BlockSpec((D_H, D_H), lambda b: (0, 0)),
            pl.BlockSpec((D_H, D_H), lambda b: (0, 0)),
            pl.BlockSpec((1, D_H), lambda b: (0, 0)),
        ],
        out_specs=[
            pl.BlockSpec((BS, D_H), lambda b: (b, 0)),
            pl.BlockSpec((BS, D_H), lambda b: (b, 0)),
            pl.BlockSpec((BS, D_H), lambda b: (b, 0)),
        ],
        out_shape=[
            jax.ShapeDtypeStruct((2 * N, D_H), jnp.float32),
            jax.ShapeDtypeStruct((2 * N, D_H), jnp.float32),
            jax.ShapeDtypeStruct((2 * N, D_H), jnp.float32),
        ],
    )(seg, cnt, cnt, q, wlt, wrt, bl)


def _mid2_body(s_ref, c0_ref, c1_ref, q_ref, wl_ref, wr_ref, bl_ref,
               h_ref, p_ref, q2_ref):
    cnt = c0_ref[0][:, :1] + c1_ref[0][:, :1]
    recip = 1.0 / jnp.maximum(cnt, 1.0)
    hp = jnp.maximum(s_ref[0] * recip + q_ref[0], 0.0)
    hn = jnp.maximum(s_ref[1] * recip + q_ref[1], 0.0)
    h_ref[0] = hp
    h_ref[1] = hn
    wl = wl_ref[...]
    p_ref[...] = jnp.concatenate(
        [jnp.dot(hp, wl, preferred_element_type=jnp.float32),
         jnp.dot(hn, wl, preferred_element_type=jnp.float32)], axis=1)
    wr = wr_ref[...]
    q2_ref[0] = (jnp.dot(hp, wr, preferred_element_type=jnp.float32)
                 + bl_ref[...])
    q2_ref[1] = (jnp.dot(hn, wr, preferred_element_type=jnp.float32)
                 + bl_ref[...])


def _tc_mid2(seg, cnt, q2d, wlt, wrt, bl):
    return pl.pallas_call(
        _mid2_body,
        grid=(NB,),
        in_specs=[
            pl.BlockSpec((NC, BS, D_H), lambda b: (0, b, 0)),
            pl.BlockSpec((1, BS, D_H), lambda b: (0, b, 0)),
            pl.BlockSpec((1, BS, D_H), lambda b: (1, b, 0)),
            pl.BlockSpec((NC, BS, D_H), lambda b: (0, b, 0)),
            pl.BlockSpec((D_H, D_OUT), lambda b: (0, 0)),
            pl.BlockSpec((D_H, D_OUT), lambda b: (0, 0)),
            pl.BlockSpec((1, D_OUT), lambda b: (0, 0)),
        ],
        out_specs=[
            pl.BlockSpec((NC, BS, D_H), lambda b: (0, b, 0)),
            pl.BlockSpec((BS, 2 * D_OUT), lambda b: (b, 0)),
            pl.BlockSpec((NC, BS, D_OUT), lambda b: (0, b, 0)),
        ],
        out_shape=[
            jax.ShapeDtypeStruct((NC, N, D_H), jnp.float32),
            jax.ShapeDtypeStruct((N, 2 * D_OUT), jnp.float32),
            jax.ShapeDtypeStruct((NC, N, D_OUT), jnp.float32),
        ],
    )(seg, cnt, cnt, q2d, wlt, wrt, bl)


def _fin_body(s_ref, c0_ref, c1_ref, q_ref, z_ref, y_ref):
    cnt = c0_ref[0][:, :1] + c1_ref[0][:, :1]
    recip = 1.0 / jnp.maximum(cnt, 1.0)
    stot = s_ref[0] + s_ref[1]
    for br in range(NC):
        z = stot[:, br * D_OUT:(br + 1) * D_OUT] * recip + q_ref[br]
        z_ref[br] = z
        m = jnp.max(z, axis=1, keepdims=True)
        e = jnp.exp(z - m)
        y_ref[br] = (z - m) - jnp.log(jnp.sum(e, axis=1, keepdims=True))


def _tc_fin(seg, cnt, q):
    return pl.pallas_call(
        _fin_body,
        grid=(NB,),
        in_specs=[
            pl.BlockSpec((NC, BS, 2 * D_OUT), lambda b: (0, b, 0)),
            pl.BlockSpec((1, BS, D_H), lambda b: (0, b, 0)),
            pl.BlockSpec((1, BS, D_H), lambda b: (1, b, 0)),
            pl.BlockSpec((NC, BS, D_OUT), lambda b: (0, b, 0)),
        ],
        out_specs=[
            pl.BlockSpec((NC, BS, D_OUT), lambda b: (0, b, 0)),
            pl.BlockSpec((NC, BS, D_OUT), lambda b: (0, b, 0)),
        ],
        out_shape=[
            jax.ShapeDtypeStruct((NC, N, D_OUT), jnp.float32),
            jax.ShapeDtypeStruct((NC, N, D_OUT), jnp.float32),
        ],
    )(seg, cnt, cnt, q)


def kernel(x, noise, Wl0, bl0, Wr0, Wl1, bl1, Wr1, Wl2, bl2, Wr2, edge_index):
    src = edge_index[0]
    dst = edge_index[1]

    pad = EPAD - E
    src_p = jnp.concatenate([src, jnp.zeros((pad,), jnp.int32)])
    dst_p = jnp.concatenate(
        [dst, N + (jnp.arange(pad, dtype=jnp.int32) % NS)])
    src2 = jnp.stack([src_p, src_p + N]).reshape(NC, NS, CPT // IB, IB, CHUNK)
    dst_t = jnp.broadcast_to(dst_p.reshape(1, NS, CPT // IB, IB, CHUNK),
                             (NC, NS, CPT // IB, IB, CHUNK))
    src_s = src_p.reshape(NC, NS, CPT2 // IB, IB, CHUNK)
    dst_s = dst_p.reshape(NC, NS, CPT2 // IB, IB, CHUNK)

    zrow = jnp.zeros((CHUNK, D_H), jnp.float32)
    ones = jnp.ones((CHUNK, D_H), jnp.float32)

    cnt = _cnt(dst_s, zrow, ones)
    p0, q0 = _tc_pre(x, noise, Wl0.T, Wr0.T, bl0.reshape(1, -1))
    s0 = _agg01(p0, src2, dst_t, zrow)
    h1, p1, q1 = _tc_mid(s0, cnt, q0, Wl1.T, Wr1.T, bl1.reshape(1, -1))
    s1 = _agg01(p1, src2, dst_t, zrow)
    h2, p2, q2 = _tc_mid2(s1, cnt, q1.reshape(NC, N, D_H),
                          Wl2.T, Wr2.T, bl2.reshape(1, -1))
    s2 = _agg2(p2, src_s, dst_s, zrow)
    z, y = _tc_fin(s2, cnt, q2)

    return (h2[0], y[0], z[0], h2[1], y[1], z[1])


# speedup vs baseline: 1.0044x; 1.0044x over previous
import functools

import jax
import jax.numpy as jnp
from jax import lax
from jax.experimental import pallas as pl
from jax.experimental.pallas import tpu as pltpu
from jax.experimental.pallas import tpu_sc as plsc

N = 10000
E = 320000
D_IN = 128
D_H = 128
D_OUT = 64

NC = 2
NS = 16

CHUNK = 128
CPT = 160
CPT2 = CPT // 2
EPAD = NS * CPT * CHUNK
NROW = 10112
RPT = NROW // NS
RCH = (128, 128, 128, 128, 120)

BS = 400
NB = N // BS
GRID = 2 * NB


def _zero_acc(z_hbm, stage_v, acc_sh, s, sem=None):
    pltpu.sync_copy(z_hbm, stage_v)
    off = 0
    cps = []
    for sz in RCH:
        dst = acc_sh.at[pl.ds(s * RPT + off, sz)]
        if sem is None:
            pltpu.sync_copy(stage_v.at[pl.ds(0, sz)], dst)
        else:
            cps.append(pltpu.async_copy(stage_v.at[pl.ds(0, sz)], dst, sem))
        off += sz
    for cp in cps:
        cp.wait()


def _copy_out(acc_sh, stage_v, o_hbm, c, s, bufs=None):
    if bufs is None:
        off = 0
        for sz in RCH:
            pltpu.sync_copy(acc_sh.at[pl.ds(s * RPT + off, sz)],
                            stage_v.at[pl.ds(0, sz)])
            pltpu.sync_copy(stage_v.at[pl.ds(0, sz)],
                            o_hbm.at[c, pl.ds(s * RPT + off, sz)])
            off += sz
        return
    rows = bufs[:2]
    sems = bufs[2:]
    hs = {}
    off = 0
    for k, sz in enumerate(RCH):
        b = k & 1
        if k >= 2:
            hs[k - 2].wait()
        pltpu.sync_copy(acc_sh.at[pl.ds(s * RPT + off, sz)],
                        rows[b].at[pl.ds(0, sz)])
        hs[k] = pltpu.async_copy(rows[b].at[pl.ds(0, sz)],
                                 o_hbm.at[c, pl.ds(s * RPT + off, sz)],
                                 sems[b])
        off += sz
    hs[len(RCH) - 2].wait()
    hs[len(RCH) - 1].wait()


IB = 8


def _make_agg(chunks, d):
    mesh = plsc.VectorSubcoreMesh(core_axis_name="c", subcore_axis_name="s")
    scratch = [
        pltpu.VMEM((IB, CHUNK), jnp.int32),
        pltpu.VMEM((IB, CHUNK), jnp.int32),
        pltpu.VMEM((CHUNK, d), jnp.float32),
        pltpu.VMEM((CHUNK, d), jnp.float32),
        pltpu.VMEM_SHARED((NROW, d), jnp.float32),
        pltpu.SemaphoreType.DMA,
        pltpu.SemaphoreType.DMA,
    ]

    @functools.partial(pl.kernel,
                       out_type=jax.ShapeDtypeStruct((NC, NROW, d),
                                                     jnp.float32),
                       mesh=mesh, scratch_types=scratch)
    def agg(p_hbm, src_hbm, dst_hbm, zrow_hbm, s_hbm,
            src_blk, dst_blk, rows0, rows1, acc_sh, sem0, sem1):
        c = lax.axis_index("c")
        s = lax.axis_index("s")
        rows = (rows0, rows1)
        sems = (sem0, sem1)

        _zero_acc(zrow_hbm, rows0, acc_sh, s, sem0)
        plsc.subcore_barrier()

        def outer(ob, carry):
            pltpu.sync_copy(src_hbm.at[c, s, ob], src_blk)
            pltpu.sync_copy(dst_hbm.at[c, s, ob], dst_blk)
            cps = [pltpu.async_copy(p_hbm.at[src_blk.at[0]], rows[0],
                                    sems[0])]
            for j in range(1, IB):
                b = j & 1
                cps.append(pltpu.async_copy(p_hbm.at[src_blk.at[j]],
                                            rows[b], sems[b]))
                cps[j - 1].wait()
                pltpu.sync_copy(rows[(j - 1) & 1],
                                acc_sh.at[dst_blk.at[j - 1]], add=True)
            cps[IB - 1].wait()
            pltpu.sync_copy(rows[(IB - 1) & 1],
                            acc_sh.at[dst_blk.at[IB - 1]], add=True)
            return carry

        lax.fori_loop(0, chunks // IB, outer, 0)
        plsc.subcore_barrier()

        _copy_out(acc_sh, rows0, s_hbm, c, s, (rows0, rows1, sem0, sem1))

    return agg


def _make_cnt():
    mesh = plsc.VectorSubcoreMesh(core_axis_name="c", subcore_axis_name="s")
    scratch = [
        pltpu.VMEM((IB, CHUNK), jnp.int32),
        pltpu.VMEM((CHUNK, D_H), jnp.float32),
        pltpu.VMEM_SHARED((NROW, D_H), jnp.float32),
        pltpu.SemaphoreType.DMA,
    ]

    @functools.partial(pl.kernel,
                       out_type=jax.ShapeDtypeStruct((NC, NROW, D_H),
                                                     jnp.float32),
                       mesh=mesh, scratch_types=scratch)
    def cntk(dst_hbm, zrow_hbm, one_hbm, cnt_hbm, dst_blk, ones_v, acc_sh,
             sem):
        c = lax.axis_index("c")
        s = lax.axis_index("s")

        _zero_acc(zrow_hbm, ones_v, acc_sh, s, sem)
        pltpu.sync_copy(one_hbm, ones_v)
        plsc.subcore_barrier()

        def outer(ob, carry):
            pltpu.sync_copy(dst_hbm.at[c, s, ob], dst_blk)
            cps = [pltpu.async_copy(ones_v, acc_sh.at[dst_blk.at[j]], sem,
                                    add=True)
                   for j in range(IB)]
            for cp in cps:
                cp.wait()
            return carry

        lax.fori_loop(0, CPT2 // IB, outer, 0)
        plsc.subcore_barrier()

        _copy_out(acc_sh, ones_v, cnt_hbm, c, s)

    return cntk


_agg01 = _make_agg(CPT, D_H)
_agg2 = _make_agg(CPT2, D_H)
_cnt = _make_cnt()


def _pre_body(x_ref, n_ref, wl_ref, wr_ref, bl_ref, p_ref, q_ref):
    b = pl.program_id(0)
    xb = x_ref[...]
    nb = n_ref[...]
    nn = jnp.sqrt(jnp.sum(nb * nb, axis=1, keepdims=True))
    noisy = xb + jnp.sign(xb) * (nb / jnp.maximum(nn, 1e-12)) * 0.3
    xx = jnp.where(b >= NB, noisy, xb)
    p_ref[...] = jnp.dot(xx, wl_ref[...], preferred_element_type=jnp.float32)
    q_ref[...] = (jnp.dot(xx, wr_ref[...], preferred_element_type=jnp.float32)
                  + bl_ref[...])


def _tc_pre(x, noise, wlt, wrt, bl):
    return pl.pallas_call(
        _pre_body,
        grid=(GRID,),
        in_specs=[
            pl.BlockSpec((BS, D_IN), lambda b: (b % NB, 0)),
            pl.BlockSpec((BS, D_IN), lambda b: (b % NB, 0)),
            pl.BlockSpec((D_IN, D_H), lambda b: (0, 0)),
            pl.BlockSpec((D_IN, D_H), lambda b: (0, 0)),
            pl.BlockSpec((1, D_H), lambda b: (0, 0)),
        ],
        out_specs=[
            pl.BlockSpec((BS, D_H), lambda b: (b, 0)),
            pl.BlockSpec((BS, D_H), lambda b: (b, 0)),
        ],
        out_shape=[
            jax.ShapeDtypeStruct((2 * N, D_H), jnp.float32),
            jax.ShapeDtypeStruct((2 * N, D_H), jnp.float32),
        ],
    )(x, noise, wlt, wrt, bl)


def _mid_body(s_ref, c0_ref, c1_ref, q_ref, wl_ref, wr_ref, bl_ref,
              h_ref, p_ref, q2_ref):
    cnt = c0_ref[0][:, :1] + c1_ref[0][:, :1]
    recip = 1.0 / jnp.maximum(cnt, 1.0)
    h = jnp.maximum(s_ref[0] * recip + q_ref[...], 0.0)
    h_ref[...] = h
    p_ref[...] = jnp.dot(h, wl_ref[...], preferred_element_type=jnp.float32)
    q2_ref[...] = (jnp.dot(h, wr_ref[...], preferred_element_type=jnp.float32)
                   + bl_ref[...])


def _tc_mid(seg, cnt, q, wlt, wrt, bl):
    return pl.pallas_call(
        _mid_body,
        grid=(GRID,),
        in_specs=[
            pl.BlockSpec((1, BS, D_H), lambda b: (b // NB, b % NB, 0)),
            pl.BlockSpec((1, BS, D_H), lambda b: (0, b % NB, 0)),
            pl.BlockSpec((1, BS, D_H), lambda b: (1, b % NB, 0)),
            pl.BlockSpec((BS, D_H), lambda b: (b, 0)),
            pl.BlockSpec((D_H, D_H), lambda b: (0, 0)),
            pl.BlockSpec((D_H, D_H), lambda b: (0, 0)),
            pl.BlockSpec((1, D_H), lambda b: (0, 0)),
        ],
        out_specs=[
            pl.BlockSpec((BS, D_H), lambda b: (b, 0)),
            pl.BlockSpec((BS, D_H), lambda b: (b, 0)),
            pl.BlockSpec((BS, D_H), lambda b: (b, 0)),
        ],
        out_shape=[
            jax.ShapeDtypeStruct((2 * N, D_H), jnp.float32),
            jax.ShapeDtypeStruct((2 * N, D_H), jnp.float32),
            jax.ShapeDtypeStruct((2 * N, D_H), jnp.float32),
        ],
    )(seg, cnt, cnt, q, wlt, wrt, bl)


def _mid2_body(s_ref, c0_ref, c1_ref, q_ref, wl_ref, wr_ref, bl_ref,
               h_ref, p_ref, q2_ref):
    cnt = c0_ref[0][:, :1] + c1_ref[0][:, :1]
    recip = 1.0 / jnp.maximum(cnt, 1.0)
    hp = jnp.maximum(s_ref[0] * recip + q_ref[0], 0.0)
    hn = jnp.maximum(s_ref[1] * recip + q_ref[1], 0.0)
    h_ref[0] = hp
    h_ref[1] = hn
    wl = wl_ref[...]
    p_ref[...] = jnp.concatenate(
        [jnp.dot(hp, wl, preferred_element_type=jnp.float32),
         jnp.dot(hn, wl, preferred_element_type=jnp.float32)], axis=1)
    wr = wr_ref[...]
    q2_ref[0] = (jnp.dot(hp, wr, preferred_element_type=jnp.float32)
                 + bl_ref[...])
    q2_ref[1] = (jnp.dot(hn, wr, preferred_element_type=jnp.float32)
                 + bl_ref[...])


def _tc_mid2(seg, cnt, q2d, wlt, wrt, bl):
    return pl.pallas_call(
        _mid2_body,
        grid=(NB,),
        in_specs=[
            pl.BlockSpec((NC, BS, D_H), lambda b: (0, b, 0)),
            pl.BlockSpec((1, BS, D_H), lambda b: (0, b, 0)),
            pl.BlockSpec((1, BS, D_H), lambda b: (1, b, 0)),
            pl.BlockSpec((NC, BS, D_H), lambda b: (0, b, 0)),
            pl.BlockSpec((D_H, D_OUT), lambda b: (0, 0)),
            pl.BlockSpec((D_H, D_OUT), lambda b: (0, 0)),
            pl.BlockSpec((1, D_OUT), lambda b: (0, 0)),
        ],
        out_specs=[
            pl.BlockSpec((NC, BS, D_H), lambda b: (0, b, 0)),
            pl.BlockSpec((BS, 2 * D_OUT), lambda b: (b, 0)),
            pl.BlockSpec((NC, BS, D_OUT), lambda b: (0, b, 0)),
        ],
        out_shape=[
            jax.ShapeDtypeStruct((NC, N, D_H), jnp.float32),
            jax.ShapeDtypeStruct((N, 2 * D_OUT), jnp.float32),
            jax.ShapeDtypeStruct((NC, N, D_OUT), jnp.float32),
        ],
    )(seg, cnt, cnt, q2d, wlt, wrt, bl)


def _fin_body(s_ref, c0_ref, c1_ref, q_ref, z_ref, y_ref):
    cnt = c0_ref[0][:, :1] + c1_ref[0][:, :1]
    recip = 1.0 / jnp.maximum(cnt, 1.0)
    stot = s_ref[0] + s_ref[1]
    for br in range(NC):
        z = stot[:, br * D_OUT:(br + 1) * D_OUT] * recip + q_ref[br]
        z_ref[br] = z
        m = jnp.max(z, axis=1, keepdims=True)
        e = jnp.exp(z - m)
        y_ref[br] = (z - m) - jnp.log(jnp.sum(e, axis=1, keepdims=True))


def _tc_fin(seg, cnt, q):
    return pl.pallas_call(
        _fin_body,
        grid=(NB,),
        in_specs=[
            pl.BlockSpec((NC, BS, 2 * D_OUT), lambda b: (0, b, 0)),
            pl.BlockSpec((1, BS, D_H), lambda b: (0, b, 0)),
            pl.BlockSpec((1, BS, D_H), lambda b: (1, b, 0)),
            pl.BlockSpec((NC, BS, D_OUT), lambda b: (0, b, 0)),
        ],
        out_specs=[
            pl.BlockSpec((NC, BS, D_OUT), lambda b: (0, b, 0)),
            pl.BlockSpec((NC, BS, D_OUT), lambda b: (0, b, 0)),
        ],
        out_shape=[
            jax.ShapeDtypeStruct((NC, N, D_OUT), jnp.float32),
            jax.ShapeDtypeStruct((NC, N, D_OUT), jnp.float32),
        ],
    )(seg, cnt, cnt, q)


def kernel(x, noise, Wl0, bl0, Wr0, Wl1, bl1, Wr1, Wl2, bl2, Wr2, edge_index):
    src = edge_index[0]
    dst = edge_index[1]

    pad = EPAD - E
    src_p = jnp.concatenate([src, jnp.zeros((pad,), jnp.int32)])
    dst_p = jnp.concatenate(
        [dst, N + (jnp.arange(pad, dtype=jnp.int32) % NS)])
    src2 = jnp.stack([src_p, src_p + N]).reshape(NC, NS, CPT // IB, IB, CHUNK)
    dst_t = jnp.broadcast_to(dst_p.reshape(1, NS, CPT // IB, IB, CHUNK),
                             (NC, NS, CPT // IB, IB, CHUNK))
    src_s = src_p.reshape(NC, NS, CPT2 // IB, IB, CHUNK)
    dst_s = dst_p.reshape(NC, NS, CPT2 // IB, IB, CHUNK)

    zrow = jnp.zeros((CHUNK, D_H), jnp.float32)
    ones = jnp.ones((CHUNK, D_H), jnp.float32)

    cnt = _cnt(dst_s, zrow, ones)
    p0, q0 = _tc_pre(x, noise, Wl0.T, Wr0.T, bl0.reshape(1, -1))
    s0 = _agg01(p0, src2, dst_t, zrow)
    h1, p1, q1 = _tc_mid(s0, cnt, q0, Wl1.T, Wr1.T, bl1.reshape(1, -1))
    s1 = _agg01(p1, src2, dst_t, zrow)
    h2, p2, q2 = _tc_mid2(s1, cnt, q1.reshape(NC, N, D_H),
                          Wl2.T, Wr2.T, bl2.reshape(1, -1))
    s2 = _agg2(p2, src_s, dst_s, zrow)
    z, y = _tc_fin(s2, cnt, q2)

    return (h2[0], y[0], z[0], h2[1], y[1], z[1])
